# baseline (speedup 1.0000x reference)
import jax
import jax.numpy as jnp
from jax.experimental import pallas as pl
from jax.experimental.pallas import tpu as pltpu

_C = 1024
_K = 8
_D = 256

_CC = 1024


def _nrm(x):
    return x * jax.lax.rsqrt(
        jnp.maximum(jnp.sum(x * x, axis=1, keepdims=True), 1e-24))


def _fused_kernel(z_ref, y_ref, p_ref, pos_ref, neg_ref, zn_s):
    j = pl.program_id(0)

    @pl.when(j == 0)
    def _zn():
        zn_s[...] = _nrm(z_ref[...]).astype(jnp.bfloat16)

    pt = _nrm(p_ref[...]).astype(jnp.bfloat16)
    pk = pt.reshape(_CC, _K, _D).swapaxes(0, 1)
    zn = zn_s[...]

    m = jax.lax.dot_general(
        pk[0], zn, dimension_numbers=(((1,), (1,)), ((), ())),
        preferred_element_type=jnp.float32).astype(jnp.bfloat16)
    for k in range(1, _K):
        m = jnp.maximum(m, jax.lax.dot_general(
            pk[k], zn, dimension_numbers=(((1,), (1,)), ((), ())),
            preferred_element_type=jnp.float32).astype(jnp.bfloat16))

    B = m.shape[1]
    cls = j * _CC + jax.lax.broadcasted_iota(jnp.int32, (_CC, B), 0)
    same = cls == y_ref[...][None, :]

    ninf = jnp.bfloat16(-jnp.inf)
    pos_c = jnp.max(jnp.where(same, m, ninf), axis=0).astype(jnp.float32)
    neg_c = jnp.max(jnp.where(same, ninf, m), axis=0).astype(jnp.float32)

    @pl.when(j == 0)
    def _init():
        pos_ref[...] = pos_c
        neg_ref[...] = neg_c

    @pl.when(j != 0)
    def _acc():
        pos_ref[...] = jnp.maximum(pos_ref[...], pos_c)
        neg_ref[...] = jnp.maximum(neg_ref[...], neg_c)


def kernel(z, y, protos):
    B = z.shape[0]
    pos, neg = pl.pallas_call(
        _fused_kernel,
        grid=(_C // _CC,),
        in_specs=[
            pl.BlockSpec((B, _D), lambda j: (0, 0)),
            pl.BlockSpec((B,), lambda j: (0,)),
            pl.BlockSpec((_CC * _K, _D), lambda j: (j, 0)),
        ],
        out_specs=[
            pl.BlockSpec((B,), lambda j: (0,)),
            pl.BlockSpec((B,), lambda j: (0,)),
        ],
        out_shape=[
            jax.ShapeDtypeStruct((B,), jnp.float32),
            jax.ShapeDtypeStruct((B,), jnp.float32),
        ],
        scratch_shapes=[pltpu.VMEM((B, _D), jnp.bfloat16)],
        compiler_params=pltpu.CompilerParams(
            dimension_semantics=("arbitrary",)),
    )(z, y, protos.reshape(_C * _K, _D))
    return (pos, neg)


# speedup vs baseline: 1.0182x; 1.0182x over previous
import jax
import jax.numpy as jnp
from jax.experimental import pallas as pl
from jax.experimental.pallas import tpu as pltpu

_C = 1024
_K = 8
_D = 256

_CC = 512


def _nrm(x):
    return x * jax.lax.rsqrt(
        jnp.maximum(jnp.sum(x * x, axis=1, keepdims=True), 1e-24))


def _fused_kernel(z_ref, y_ref, p_ref, pos_ref, neg_ref, zn_s):
    j = pl.program_id(0)

    @pl.when(j == 0)
    def _zn():
        zn_s[...] = _nrm(z_ref[...]).astype(jnp.bfloat16)

    pt = _nrm(p_ref[...]).astype(jnp.bfloat16)
    pk = pt.reshape(_CC, _K, _D).swapaxes(0, 1)
    zn = zn_s[...]

    m = jax.lax.dot_general(
        pk[0], zn, dimension_numbers=(((1,), (1,)), ((), ())),
        preferred_element_type=jnp.float32).astype(jnp.bfloat16)
    for k in range(1, _K):
        m = jnp.maximum(m, jax.lax.dot_general(
            pk[k], zn, dimension_numbers=(((1,), (1,)), ((), ())),
            preferred_element_type=jnp.float32).astype(jnp.bfloat16))

    B = m.shape[1]
    cls = j * _CC + jax.lax.broadcasted_iota(jnp.int32, (_CC, B), 0)
    same = cls == y_ref[...][None, :]

    ninf = jnp.bfloat16(-jnp.inf)
    pos_c = jnp.max(jnp.where(same, m, ninf), axis=0).astype(jnp.float32)
    neg_c = jnp.max(jnp.where(same, ninf, m), axis=0).astype(jnp.float32)

    @pl.when(j == 0)
    def _init():
        pos_ref[...] = pos_c
        neg_ref[...] = neg_c

    @pl.when(j != 0)
    def _acc():
        pos_ref[...] = jnp.maximum(pos_ref[...], pos_c)
        neg_ref[...] = jnp.maximum(neg_ref[...], neg_c)


def kernel(z, y, protos):
    B = z.shape[0]
    pos, neg = pl.pallas_call(
        _fused_kernel,
        grid=(_C // _CC,),
        in_specs=[
            pl.BlockSpec((B, _D), lambda j: (0, 0)),
            pl.BlockSpec((B,), lambda j: (0,)),
            pl.BlockSpec((_CC * _K, _D), lambda j: (j, 0)),
        ],
        out_specs=[
            pl.BlockSpec((B,), lambda j: (0,)),
            pl.BlockSpec((B,), lambda j: (0,)),
        ],
        out_shape=[
            jax.ShapeDtypeStruct((B,), jnp.float32),
            jax.ShapeDtypeStruct((B,), jnp.float32),
        ],
        scratch_shapes=[pltpu.VMEM((B, _D), jnp.bfloat16)],
        compiler_params=pltpu.CompilerParams(
            dimension_semantics=("arbitrary",)),
    )(z, y, protos.reshape(_C * _K, _D))
    return (pos, neg)
